# baseline (speedup 1.0000x reference)
import jax
import jax.numpy as jnp
from jax import lax
from jax.experimental import pallas as pl
from jax.experimental.pallas import tpu as pltpu
from jax.experimental.pallas import tpu_sc as plsc

BATCH = 4096
HIST = 50
EMBED_DIM = 32
VOCAB = 1000000

NC = 2
NS = 16
NW = NC * NS

BAGS_PER_W = BATCH // NW
BAGS_PER_STEP = 2
IDX_PER_STEP = BAGS_PER_STEP * HIST
STEPS = BAGS_PER_W // BAGS_PER_STEP
NBUF = 2
IDX_W = 128

D2 = EMBED_DIM // 2
UNITS = VOCAB // 4
TCOLS = VOCAB // 128
TAIL = VOCAB - TCOLS * 128
CW = 256
PSTEPS = (TCOLS * 128) // CW
PBUF = 4
TPW = 124


def _transpose_block(src, dst, vv_count, lanes):
    for h in range(vv_count // D2):
        vv = h * D2 + lanes
        r16 = vv // 4
        for d in range(EMBED_DIM):
            c16 = (vv % 4) * EMBED_DIM + d
            plsc.store_scatter(dst, [r16, c16], src[d, pl.ds(h * D2, D2)])


def _pack_body(tt_hbm, t4_hbm, ins, outs, tl_in, tl_out, sis, sos, stail):
    c = lax.axis_index("c")
    s = lax.axis_index("s")
    wid = s * NC + c
    base = wid * TPW
    lanes = lax.iota(jnp.int32, D2)
    orows = CW // 4

    def start_in(i, b):
        tc = base + i

        @pl.when((tc < PSTEPS) & (i < TPW))
        def _():
            pltpu.async_copy(tt_hbm.at[:, pl.ds(tc * CW, CW)], ins[b],
                             sis[b])

    for b in range(PBUF):
        start_in(b, b)

    def outer(o, carry):
        for b in range(PBUF):
            i = o * PBUF + b
            tc = base + i
            live = tc < PSTEPS

            @pl.when(live)
            def _():
                pltpu.make_async_copy(
                    tt_hbm.at[:, pl.ds(tc * CW, CW)], ins[b], sis[b]).wait()

            @pl.when(live & (i >= PBUF))
            def _():
                pltpu.make_async_copy(
                    outs[b], t4_hbm.at[pl.ds(tc * orows, orows)],
                    sos[b]).wait()

            @pl.when(live)
            def _():
                _transpose_block(ins[b], outs[b], CW, lanes)
                pltpu.async_copy(
                    outs[b], t4_hbm.at[pl.ds(tc * orows, orows)], sos[b])
                start_in(i + PBUF, b)
        return carry

    lax.fori_loop(0, TPW // PBUF, outer, 0)

    n_live = jnp.clip(PSTEPS - base, 0, TPW)
    for b in range(PBUF):
        @pl.when(n_live > b)
        def _():
            pltpu.make_async_copy(
                outs[b], t4_hbm.at[pl.ds(0, orows)], sos[b]).wait()

    @pl.when(wid == NW - 1)
    def _():
        pltpu.async_copy(tt_hbm.at[:, pl.ds(TCOLS * 128, TAIL)], tl_in, stail)
        pltpu.make_async_copy(tt_hbm.at[:, pl.ds(TCOLS * 128, TAIL)], tl_in,
                              stail).wait()
        _transpose_block(tl_in, tl_out, TAIL, lanes)
        pltpu.async_copy(tl_out, t4_hbm.at[pl.ds(TCOLS * 32, TAIL // 4)],
                         stail)
        pltpu.make_async_copy(tl_out, t4_hbm.at[pl.ds(TCOLS * 32, TAIL // 4)],
                              stail).wait()


def _gather_body(tbl_hbm, u_hbm, q_hbm, out_hbm, u_v, q_v, rows0, rows1,
                 out_v, sem0, sem1):
    c = lax.axis_index("c")
    s = lax.axis_index("s")
    wid = s * NC + c
    ibase = wid * STEPS
    obase = wid * BAGS_PER_W

    pltpu.sync_copy(u_hbm.at[pl.ds(ibase, STEPS)], u_v)
    pltpu.sync_copy(q_hbm.at[pl.ds(ibase, STEPS)], q_v)

    rows = (rows0, rows1)
    sems = (sem0, sem1)

    def gather(step, buf):
        pltpu.async_copy(
            tbl_hbm.at[u_v.at[step, pl.ds(0, IDX_PER_STEP)]], rows[buf],
            sems[buf])

    gather(0, 0)
    gather(1, 1)

    lanes = lax.iota(jnp.int32, D2)

    def reduce_step(buf, step):
        rb = rows[buf]
        step_vec = jnp.full((D2,), step, jnp.int32)

        def sub_row(j):
            qb = plsc.load_gather(q_v,
                                  [step_vec, jnp.full((D2,), j, jnp.int32)])
            row = jnp.full((D2,), j, jnp.int32)
            lo = plsc.load_gather(rb, [row, qb + lanes])
            hi = plsc.load_gather(rb, [row, qb + (D2 + lanes)])
            return lo, hi

        for r in range(BAGS_PER_STEP):
            off = r * HIST
            lo, hi = sub_row(off)
            for j in range(1, HIST):
                l2, h2 = sub_row(off + j)
                lo = lo + l2
                hi = hi + h2
            orow = step * BAGS_PER_STEP + r
            out_v[orow, pl.ds(0, D2)] = lo
            out_v[orow, pl.ds(D2, D2)] = hi

    def outer(o, carry):
        for b in range(NBUF):
            step = o * NBUF + b
            pltpu.make_async_copy(
                tbl_hbm.at[u_v.at[step, pl.ds(0, IDX_PER_STEP)]], rows[b],
                sems[b]).wait()

            @pl.when(step + NBUF < STEPS)
            def _():
                gather(step + NBUF, b)

            reduce_step(b, step)
        return carry

    lax.fori_loop(0, STEPS // NBUF, outer, 0)

    pltpu.sync_copy(out_v, out_hbm.at[pl.ds(obase, BAGS_PER_W)])


@jax.jit
def _embedding_bag(inputs, table):
    v = inputs.astype(jnp.int32).reshape(NW * STEPS, IDX_PER_STEP)
    pad = ((0, 0), (0, IDX_W - IDX_PER_STEP))
    u = jnp.pad(v >> 2, pad)
    q = jnp.pad((v & 3) * EMBED_DIM, pad)

    mesh = plsc.VectorSubcoreMesh(core_axis_name="c", subcore_axis_name="s")
    params = pltpu.CompilerParams(needs_layout_passes=False)

    pack = pl.kernel(
        _pack_body,
        out_type=jax.ShapeDtypeStruct((UNITS, IDX_W), jnp.float32),
        mesh=mesh,
        compiler_params=params,
        scratch_types=[
            [pltpu.VMEM((EMBED_DIM, CW), jnp.float32) for _ in range(PBUF)],
            [pltpu.VMEM((CW // 4, IDX_W), jnp.float32) for _ in range(PBUF)],
            pltpu.VMEM((EMBED_DIM, TAIL), jnp.float32),
            pltpu.VMEM((TAIL // 4, IDX_W), jnp.float32),
            [pltpu.SemaphoreType.DMA for _ in range(PBUF)],
            [pltpu.SemaphoreType.DMA for _ in range(PBUF)],
            pltpu.SemaphoreType.DMA,
        ],
    )
    t4 = pack(table.T)

    run = pl.kernel(
        _gather_body,
        out_type=jax.ShapeDtypeStruct((BATCH, IDX_W), jnp.float32),
        mesh=mesh,
        compiler_params=params,
        scratch_types=[
            pltpu.VMEM((STEPS, IDX_W), jnp.int32),
            pltpu.VMEM((STEPS, IDX_W), jnp.int32),
            pltpu.VMEM((IDX_PER_STEP, IDX_W), jnp.float32),
            pltpu.VMEM((IDX_PER_STEP, IDX_W), jnp.float32),
            pltpu.VMEM((BAGS_PER_W, IDX_W), jnp.float32),
            pltpu.SemaphoreType.DMA,
            pltpu.SemaphoreType.DMA,
        ],
    )
    return run(t4, u, q)[:, :EMBED_DIM]


def kernel(inputs, table):
    return _embedding_bag(inputs, table)


# speedup vs baseline: 1.2458x; 1.2458x over previous
import jax
import jax.numpy as jnp
from jax import lax
from jax.experimental import pallas as pl
from jax.experimental.pallas import tpu as pltpu
from jax.experimental.pallas import tpu_sc as plsc

BATCH = 4096
HIST = 50
EMBED_DIM = 32
VOCAB = 1000000

NC = 2
NS = 16
NW = NC * NS

BAGS_PER_W = BATCH // NW
BAGS_PER_STEP = 2
IDX_PER_STEP = BAGS_PER_STEP * HIST
STEPS = BAGS_PER_W // BAGS_PER_STEP
NBUF = 2
IDX_W = 128

D2 = EMBED_DIM // 2
UNITS = VOCAB // 4
TCOLS = VOCAB // 128
TAIL = VOCAB - TCOLS * 128
CW = 256
PSTEPS = (TCOLS * 128) // CW
PBUF = 4
TPW = 124


def _transpose_block(src, dst, vv_count, lanes):
    base_r = lanes // 4
    qpat = (lanes % 4) * EMBED_DIM
    for h in range(vv_count // D2):
        r16 = base_r + 4 * h
        vals = [src[d, pl.ds(h * D2, D2)] for d in range(EMBED_DIM)]
        for d in range(EMBED_DIM):
            plsc.store_scatter(dst, [r16, qpat + d], vals[d])


def _pack_body(tt_hbm, t4_hbm, ins, outs, tl_in, tl_out, sis, sos, stail):
    c = lax.axis_index("c")
    s = lax.axis_index("s")
    wid = s * NC + c
    base = wid * TPW
    lanes = lax.iota(jnp.int32, D2)
    orows = CW // 4

    def start_in(i, b):
        tc = base + i

        @pl.when((tc < PSTEPS) & (i < TPW))
        def _():
            pltpu.async_copy(tt_hbm.at[:, pl.ds(tc * CW, CW)], ins[b],
                             sis[b])

    for b in range(PBUF):
        start_in(b, b)

    def outer(o, carry):
        for b in range(PBUF):
            i = o * PBUF + b
            tc = base + i
            live = tc < PSTEPS

            @pl.when(live)
            def _():
                pltpu.make_async_copy(
                    tt_hbm.at[:, pl.ds(tc * CW, CW)], ins[b], sis[b]).wait()

            @pl.when(live & (i >= PBUF))
            def _():
                pltpu.make_async_copy(
                    outs[b], t4_hbm.at[pl.ds(tc * orows, orows)],
                    sos[b]).wait()

            @pl.when(live)
            def _():
                _transpose_block(ins[b], outs[b], CW, lanes)
                pltpu.async_copy(
                    outs[b], t4_hbm.at[pl.ds(tc * orows, orows)], sos[b])
                start_in(i + PBUF, b)
        return carry

    lax.fori_loop(0, TPW // PBUF, outer, 0)

    n_live = jnp.clip(PSTEPS - base, 0, TPW)
    for b in range(PBUF):
        @pl.when(n_live > b)
        def _():
            pltpu.make_async_copy(
                outs[b], t4_hbm.at[pl.ds(0, orows)], sos[b]).wait()

    @pl.when(wid == NW - 1)
    def _():
        pltpu.async_copy(tt_hbm.at[:, pl.ds(TCOLS * 128, TAIL)], tl_in, stail)
        pltpu.make_async_copy(tt_hbm.at[:, pl.ds(TCOLS * 128, TAIL)], tl_in,
                              stail).wait()
        _transpose_block(tl_in, tl_out, TAIL, lanes)
        pltpu.async_copy(tl_out, t4_hbm.at[pl.ds(TCOLS * 32, TAIL // 4)],
                         stail)
        pltpu.make_async_copy(tl_out, t4_hbm.at[pl.ds(TCOLS * 32, TAIL // 4)],
                              stail).wait()


def _gather_body(tbl_hbm, u_hbm, q_hbm, out_hbm, u_v, q_v, rows0, rows1,
                 out_v, sem0, sem1):
    c = lax.axis_index("c")
    s = lax.axis_index("s")
    wid = s * NC + c
    ibase = wid * STEPS
    obase = wid * BAGS_PER_W

    pltpu.sync_copy(u_hbm.at[pl.ds(ibase, STEPS)], u_v)
    pltpu.sync_copy(q_hbm.at[pl.ds(ibase, STEPS)], q_v)

    rows = (rows0, rows1)
    sems = (sem0, sem1)

    def gather(step, buf):
        pltpu.async_copy(
            tbl_hbm.at[u_v.at[step, pl.ds(0, IDX_PER_STEP)]], rows[buf],
            sems[buf])

    gather(0, 0)
    gather(1, 1)

    lanes = lax.iota(jnp.int32, D2)

    def reduce_step(buf, step):
        rb = rows[buf]
        step_vec = jnp.full((D2,), step, jnp.int32)

        def sub_row(j):
            qb = plsc.load_gather(q_v,
                                  [step_vec, jnp.full((D2,), j, jnp.int32)])
            row = jnp.full((D2,), j, jnp.int32)
            lo = plsc.load_gather(rb, [row, qb + lanes])
            hi = plsc.load_gather(rb, [row, qb + (D2 + lanes)])
            return lo, hi

        for r in range(BAGS_PER_STEP):
            off = r * HIST
            lo, hi = sub_row(off)
            for j in range(1, HIST):
                l2, h2 = sub_row(off + j)
                lo = lo + l2
                hi = hi + h2
            orow = step * BAGS_PER_STEP + r
            out_v[orow, pl.ds(0, D2)] = lo
            out_v[orow, pl.ds(D2, D2)] = hi

    def outer(o, carry):
        for b in range(NBUF):
            step = o * NBUF + b
            pltpu.make_async_copy(
                tbl_hbm.at[u_v.at[step, pl.ds(0, IDX_PER_STEP)]], rows[b],
                sems[b]).wait()

            @pl.when(step + NBUF < STEPS)
            def _():
                gather(step + NBUF, b)

            reduce_step(b, step)
        return carry

    lax.fori_loop(0, STEPS // NBUF, outer, 0)

    pltpu.sync_copy(out_v, out_hbm.at[pl.ds(obase, BAGS_PER_W)])


@jax.jit
def _embedding_bag(inputs, table):
    v = inputs.astype(jnp.int32).reshape(NW * STEPS, IDX_PER_STEP)
    pad = ((0, 0), (0, IDX_W - IDX_PER_STEP))
    u = jnp.pad(v >> 2, pad)
    q = jnp.pad((v & 3) * EMBED_DIM, pad)

    mesh = plsc.VectorSubcoreMesh(core_axis_name="c", subcore_axis_name="s")
    params = pltpu.CompilerParams(needs_layout_passes=False)

    pack = pl.kernel(
        _pack_body,
        out_type=jax.ShapeDtypeStruct((UNITS, IDX_W), jnp.float32),
        mesh=mesh,
        compiler_params=params,
        scratch_types=[
            [pltpu.VMEM((EMBED_DIM, CW), jnp.float32) for _ in range(PBUF)],
            [pltpu.VMEM((CW // 4, IDX_W), jnp.float32) for _ in range(PBUF)],
            pltpu.VMEM((EMBED_DIM, TAIL), jnp.float32),
            pltpu.VMEM((TAIL // 4, IDX_W), jnp.float32),
            [pltpu.SemaphoreType.DMA for _ in range(PBUF)],
            [pltpu.SemaphoreType.DMA for _ in range(PBUF)],
            pltpu.SemaphoreType.DMA,
        ],
    )
    t4 = pack(table.T)

    run = pl.kernel(
        _gather_body,
        out_type=jax.ShapeDtypeStruct((BATCH, IDX_W), jnp.float32),
        mesh=mesh,
        compiler_params=params,
        scratch_types=[
            pltpu.VMEM((STEPS, IDX_W), jnp.int32),
            pltpu.VMEM((STEPS, IDX_W), jnp.int32),
            pltpu.VMEM((IDX_PER_STEP, IDX_W), jnp.float32),
            pltpu.VMEM((IDX_PER_STEP, IDX_W), jnp.float32),
            pltpu.VMEM((BAGS_PER_W, IDX_W), jnp.float32),
            pltpu.SemaphoreType.DMA,
            pltpu.SemaphoreType.DMA,
        ],
    )
    return run(t4, u, q)[:, :EMBED_DIM]


def kernel(inputs, table):
    return _embedding_bag(inputs, table)


# speedup vs baseline: 1.2686x; 1.0183x over previous
import jax
import jax.numpy as jnp
from jax import lax
from jax.experimental import pallas as pl
from jax.experimental.pallas import tpu as pltpu
from jax.experimental.pallas import tpu_sc as plsc

BATCH = 4096
HIST = 50
EMBED_DIM = 32
VOCAB = 1000000

NC = 2
NS = 16
NW = NC * NS

BAGS_PER_W = BATCH // NW
BAGS_PER_STEP = 2
IDX_PER_STEP = BAGS_PER_STEP * HIST
STEPS = BAGS_PER_W // BAGS_PER_STEP
NBUF = 2
IDX_W = 128

D2 = EMBED_DIM // 2
UNITS = VOCAB // 4
TCOLS = VOCAB // 128
TAIL = VOCAB - TCOLS * 128
CW = 256
PSTEPS = (TCOLS * 128) // CW
PBUF = 4
TPW = 124


def _transpose_block(src, dst, vv_count, lanes):
    base_r = lanes // 4
    qpat = (lanes % 4) * EMBED_DIM

    @plsc.parallel_loop(0, vv_count // D2, unroll=4)
    def _(h):
        r16 = base_r + 4 * h
        vals = [src[d, pl.ds(h * D2, D2)] for d in range(EMBED_DIM)]
        for d in range(EMBED_DIM):
            plsc.store_scatter(dst, [r16, qpat + d], vals[d])


def _pack_body(tt_hbm, t4_hbm, ins, outs, tl_in, tl_out, sis, sos, stail):
    c = lax.axis_index("c")
    s = lax.axis_index("s")
    wid = s * NC + c
    base = wid * TPW
    lanes = lax.iota(jnp.int32, D2)
    orows = CW // 4

    def start_in(i, b):
        tc = base + i

        @pl.when((tc < PSTEPS) & (i < TPW))
        def _():
            pltpu.async_copy(tt_hbm.at[:, pl.ds(tc * CW, CW)], ins[b],
                             sis[b])

    for b in range(PBUF):
        start_in(b, b)

    def outer(o, carry):
        for b in range(PBUF):
            i = o * PBUF + b
            tc = base + i
            live = tc < PSTEPS

            @pl.when(live)
            def _():
                pltpu.make_async_copy(
                    tt_hbm.at[:, pl.ds(tc * CW, CW)], ins[b], sis[b]).wait()

            @pl.when(live & (i >= PBUF))
            def _():
                pltpu.make_async_copy(
                    outs[b], t4_hbm.at[pl.ds(tc * orows, orows)],
                    sos[b]).wait()

            @pl.when(live)
            def _():
                _transpose_block(ins[b], outs[b], CW, lanes)
                pltpu.async_copy(
                    outs[b], t4_hbm.at[pl.ds(tc * orows, orows)], sos[b])
                start_in(i + PBUF, b)
        return carry

    lax.fori_loop(0, TPW // PBUF, outer, 0)

    n_live = jnp.clip(PSTEPS - base, 0, TPW)
    for b in range(PBUF):
        @pl.when(n_live > b)
        def _():
            pltpu.make_async_copy(
                outs[b], t4_hbm.at[pl.ds(0, orows)], sos[b]).wait()

    @pl.when(wid == NW - 1)
    def _():
        pltpu.async_copy(tt_hbm.at[:, pl.ds(TCOLS * 128, TAIL)], tl_in, stail)
        pltpu.make_async_copy(tt_hbm.at[:, pl.ds(TCOLS * 128, TAIL)], tl_in,
                              stail).wait()
        _transpose_block(tl_in, tl_out, TAIL, lanes)
        pltpu.async_copy(tl_out, t4_hbm.at[pl.ds(TCOLS * 32, TAIL // 4)],
                         stail)
        pltpu.make_async_copy(tl_out, t4_hbm.at[pl.ds(TCOLS * 32, TAIL // 4)],
                              stail).wait()


def _gather_body(tbl_hbm, u_hbm, q_hbm, out_hbm, u_v, q_v, rows0, rows1,
                 out_v, sem0, sem1):
    c = lax.axis_index("c")
    s = lax.axis_index("s")
    wid = s * NC + c
    ibase = wid * STEPS
    obase = wid * BAGS_PER_W

    pltpu.sync_copy(u_hbm.at[pl.ds(ibase, STEPS)], u_v)
    pltpu.sync_copy(q_hbm.at[pl.ds(ibase, STEPS)], q_v)

    rows = (rows0, rows1)
    sems = (sem0, sem1)

    def gather(step, buf):
        pltpu.async_copy(
            tbl_hbm.at[u_v.at[step, pl.ds(0, IDX_PER_STEP)]], rows[buf],
            sems[buf])

    gather(0, 0)
    gather(1, 1)

    lanes = lax.iota(jnp.int32, D2)

    def reduce_step(buf, step):
        rb = rows[buf]
        step_vec = jnp.full((D2,), step, jnp.int32)

        def sub_row(j):
            qb = plsc.load_gather(q_v,
                                  [step_vec, jnp.full((D2,), j, jnp.int32)])
            row = jnp.full((D2,), j, jnp.int32)
            lo = plsc.load_gather(rb, [row, qb + lanes])
            hi = plsc.load_gather(rb, [row, qb + (D2 + lanes)])
            return lo, hi

        for r in range(BAGS_PER_STEP):
            off = r * HIST
            lo, hi = sub_row(off)
            for j in range(1, HIST):
                l2, h2 = sub_row(off + j)
                lo = lo + l2
                hi = hi + h2
            orow = step * BAGS_PER_STEP + r
            out_v[orow, pl.ds(0, D2)] = lo
            out_v[orow, pl.ds(D2, D2)] = hi

    def outer(o, carry):
        for b in range(NBUF):
            step = o * NBUF + b
            pltpu.make_async_copy(
                tbl_hbm.at[u_v.at[step, pl.ds(0, IDX_PER_STEP)]], rows[b],
                sems[b]).wait()

            @pl.when(step + NBUF < STEPS)
            def _():
                gather(step + NBUF, b)

            reduce_step(b, step)
        return carry

    lax.fori_loop(0, STEPS // NBUF, outer, 0)

    pltpu.sync_copy(out_v, out_hbm.at[pl.ds(obase, BAGS_PER_W)])


@jax.jit
def _embedding_bag(inputs, table):
    v = inputs.astype(jnp.int32).reshape(NW * STEPS, IDX_PER_STEP)
    pad = ((0, 0), (0, IDX_W - IDX_PER_STEP))
    u = jnp.pad(v >> 2, pad)
    q = jnp.pad((v & 3) * EMBED_DIM, pad)

    mesh = plsc.VectorSubcoreMesh(core_axis_name="c", subcore_axis_name="s")
    params = pltpu.CompilerParams(needs_layout_passes=False)

    pack = pl.kernel(
        _pack_body,
        out_type=jax.ShapeDtypeStruct((UNITS, IDX_W), jnp.float32),
        mesh=mesh,
        compiler_params=params,
        scratch_types=[
            [pltpu.VMEM((EMBED_DIM, CW), jnp.float32) for _ in range(PBUF)],
            [pltpu.VMEM((CW // 4, IDX_W), jnp.float32) for _ in range(PBUF)],
            pltpu.VMEM((EMBED_DIM, TAIL), jnp.float32),
            pltpu.VMEM((TAIL // 4, IDX_W), jnp.float32),
            [pltpu.SemaphoreType.DMA for _ in range(PBUF)],
            [pltpu.SemaphoreType.DMA for _ in range(PBUF)],
            pltpu.SemaphoreType.DMA,
        ],
    )
    t4 = pack(table.T)

    run = pl.kernel(
        _gather_body,
        out_type=jax.ShapeDtypeStruct((BATCH, IDX_W), jnp.float32),
        mesh=mesh,
        compiler_params=params,
        scratch_types=[
            pltpu.VMEM((STEPS, IDX_W), jnp.int32),
            pltpu.VMEM((STEPS, IDX_W), jnp.int32),
            pltpu.VMEM((IDX_PER_STEP, IDX_W), jnp.float32),
            pltpu.VMEM((IDX_PER_STEP, IDX_W), jnp.float32),
            pltpu.VMEM((BAGS_PER_W, IDX_W), jnp.float32),
            pltpu.SemaphoreType.DMA,
            pltpu.SemaphoreType.DMA,
        ],
    )
    return run(t4, u, q)[:, :EMBED_DIM]


def kernel(inputs, table):
    return _embedding_bag(inputs, table)


# speedup vs baseline: 2.5194x; 1.9859x over previous
import jax
import jax.numpy as jnp
from jax import lax
from jax.experimental import pallas as pl
from jax.experimental.pallas import tpu as pltpu
from jax.experimental.pallas import tpu_sc as plsc

BATCH = 4096
HIST = 50
EMBED_DIM = 32
VOCAB = 1000000

NC = 2
NS = 16
NW = NC * NS

BAGS_PER_W = BATCH // NW
BAGS_PER_STEP = 2
IDX_PER_STEP = BAGS_PER_STEP * HIST
STEPS = BAGS_PER_W // BAGS_PER_STEP
NBUF = 2
IDX_W = 128

D2 = EMBED_DIM // 2
UNITS = VOCAB // 4
TCOLS = VOCAB // 128
TAIL = VOCAB - TCOLS * 128
CW = 128
PSTEPS = (TCOLS * 128) // CW
PBUF = 4
TPW = 248


def _transpose_block(src, dst, vv_count, lanes):
    for v0 in range(0, vv_count, 8):
        items = []
        for k in range(8):
            vvm = (v0 + k + lanes) % vv_count
            items.append((vvm,
                          plsc.load_gather(src, [lanes, vvm]),
                          plsc.load_gather(src, [lanes + D2, vvm])))
        for vvm, lo, hi in items:
            r = vvm // 4
            c = (vvm % 4) * EMBED_DIM + lanes
            plsc.store_scatter(dst, [r, c], lo)
            plsc.store_scatter(dst, [r, c + D2], hi)


def _pack_body(tt_hbm, t4_hbm, ins, outs, tl_in, tl_out, sis, sos, stail):
    c = lax.axis_index("c")
    s = lax.axis_index("s")
    wid = s * NC + c
    base = wid * TPW
    lanes = lax.iota(jnp.int32, D2)
    orows = CW // 4

    def start_in(i, b):
        tc = base + i

        @pl.when((tc < PSTEPS) & (i < TPW))
        def _():
            pltpu.async_copy(tt_hbm.at[:, pl.ds(tc * CW, CW)], ins[b],
                             sis[b])

    for b in range(PBUF):
        start_in(b, b)

    def outer(o, carry):
        for b in range(PBUF):
            i = o * PBUF + b
            tc = base + i
            live = tc < PSTEPS

            @pl.when(live)
            def _():
                pltpu.make_async_copy(
                    tt_hbm.at[:, pl.ds(tc * CW, CW)], ins[b], sis[b]).wait()

            @pl.when(live & (i >= PBUF))
            def _():
                pltpu.make_async_copy(
                    outs[b], t4_hbm.at[pl.ds(tc * orows, orows)],
                    sos[b]).wait()

            @pl.when(live)
            def _():
                _transpose_block(ins[b], outs[b], CW, lanes)
                pltpu.async_copy(
                    outs[b], t4_hbm.at[pl.ds(tc * orows, orows)], sos[b])
                start_in(i + PBUF, b)
        return carry

    lax.fori_loop(0, TPW // PBUF, outer, 0)

    n_live = jnp.clip(PSTEPS - base, 0, TPW)
    for b in range(PBUF):
        @pl.when(n_live > b)
        def _():
            pltpu.make_async_copy(
                outs[b], t4_hbm.at[pl.ds(0, orows)], sos[b]).wait()

    @pl.when(wid == NW - 1)
    def _():
        pltpu.async_copy(tt_hbm.at[:, pl.ds(TCOLS * 128, TAIL)], tl_in, stail)
        pltpu.make_async_copy(tt_hbm.at[:, pl.ds(TCOLS * 128, TAIL)], tl_in,
                              stail).wait()
        _transpose_block(tl_in, tl_out, TAIL, lanes)
        pltpu.async_copy(tl_out, t4_hbm.at[pl.ds(TCOLS * 32, TAIL // 4)],
                         stail)
        pltpu.make_async_copy(tl_out, t4_hbm.at[pl.ds(TCOLS * 32, TAIL // 4)],
                              stail).wait()


def _gather_body(tbl_hbm, u_hbm, q_hbm, out_hbm, u_v, q_v, rows0, rows1,
                 out_v, sem0, sem1):
    c = lax.axis_index("c")
    s = lax.axis_index("s")
    wid = s * NC + c
    ibase = wid * STEPS
    obase = wid * BAGS_PER_W

    pltpu.sync_copy(u_hbm.at[pl.ds(ibase, STEPS)], u_v)
    pltpu.sync_copy(q_hbm.at[pl.ds(ibase, STEPS)], q_v)

    rows = (rows0, rows1)
    sems = (sem0, sem1)

    def gather(step, buf):
        pltpu.async_copy(
            tbl_hbm.at[u_v.at[step, pl.ds(0, IDX_PER_STEP)]], rows[buf],
            sems[buf])

    gather(0, 0)
    gather(1, 1)

    lanes = lax.iota(jnp.int32, D2)

    def reduce_step(buf, step):
        rb = rows[buf]
        step_vec = jnp.full((D2,), step, jnp.int32)

        def sub_row(j):
            qb = plsc.load_gather(q_v,
                                  [step_vec, jnp.full((D2,), j, jnp.int32)])
            row = jnp.full((D2,), j, jnp.int32)
            lo = plsc.load_gather(rb, [row, qb + lanes])
            hi = plsc.load_gather(rb, [row, qb + (D2 + lanes)])
            return lo, hi

        for r in range(BAGS_PER_STEP):
            off = r * HIST
            lo, hi = sub_row(off)
            for j in range(1, HIST):
                l2, h2 = sub_row(off + j)
                lo = lo + l2
                hi = hi + h2
            orow = step * BAGS_PER_STEP + r
            out_v[orow, pl.ds(0, D2)] = lo
            out_v[orow, pl.ds(D2, D2)] = hi

    def outer(o, carry):
        for b in range(NBUF):
            step = o * NBUF + b
            pltpu.make_async_copy(
                tbl_hbm.at[u_v.at[step, pl.ds(0, IDX_PER_STEP)]], rows[b],
                sems[b]).wait()

            @pl.when(step + NBUF < STEPS)
            def _():
                gather(step + NBUF, b)

            reduce_step(b, step)
        return carry

    lax.fori_loop(0, STEPS // NBUF, outer, 0)

    pltpu.sync_copy(out_v, out_hbm.at[pl.ds(obase, BAGS_PER_W)])


@jax.jit
def _embedding_bag(inputs, table):
    v = inputs.astype(jnp.int32).reshape(NW * STEPS, IDX_PER_STEP)
    pad = ((0, 0), (0, IDX_W - IDX_PER_STEP))
    u = jnp.pad(v >> 2, pad)
    q = jnp.pad((v & 3) * EMBED_DIM, pad)

    mesh = plsc.VectorSubcoreMesh(core_axis_name="c", subcore_axis_name="s")
    params = pltpu.CompilerParams(needs_layout_passes=False)

    pack = pl.kernel(
        _pack_body,
        out_type=jax.ShapeDtypeStruct((UNITS, IDX_W), jnp.float32),
        mesh=mesh,
        compiler_params=params,
        scratch_types=[
            [pltpu.VMEM((EMBED_DIM, CW), jnp.float32) for _ in range(PBUF)],
            [pltpu.VMEM((CW // 4, IDX_W), jnp.float32) for _ in range(PBUF)],
            pltpu.VMEM((EMBED_DIM, TAIL), jnp.float32),
            pltpu.VMEM((TAIL // 4, IDX_W), jnp.float32),
            [pltpu.SemaphoreType.DMA for _ in range(PBUF)],
            [pltpu.SemaphoreType.DMA for _ in range(PBUF)],
            pltpu.SemaphoreType.DMA,
        ],
    )
    t4 = pack(table.T)

    run = pl.kernel(
        _gather_body,
        out_type=jax.ShapeDtypeStruct((BATCH, IDX_W), jnp.float32),
        mesh=mesh,
        compiler_params=params,
        scratch_types=[
            pltpu.VMEM((STEPS, IDX_W), jnp.int32),
            pltpu.VMEM((STEPS, IDX_W), jnp.int32),
            pltpu.VMEM((IDX_PER_STEP, IDX_W), jnp.float32),
            pltpu.VMEM((IDX_PER_STEP, IDX_W), jnp.float32),
            pltpu.VMEM((BAGS_PER_W, IDX_W), jnp.float32),
            pltpu.SemaphoreType.DMA,
            pltpu.SemaphoreType.DMA,
        ],
    )
    return run(t4, u, q)[:, :EMBED_DIM]


def kernel(inputs, table):
    return _embedding_bag(inputs, table)
